# baseline (speedup 1.0000x reference)
import functools

import jax
import jax.numpy as jnp
from jax import lax
from jax.experimental import pallas as pl
from jax.experimental.pallas import tpu as pltpu
from jax.experimental.pallas import tpu_sc as plsc

_VOCAB = 1_000_000
_N_EMBD = 64
_BATCH = 4096
_SEQ = 200
_B = _BATCH * _SEQ
_NC, _NS = 2, 16
_NW = _NC * _NS
_PER_W = _B // _NW
_CHUNK = 1024
_N_CHUNKS = _PER_W // _CHUNK


@functools.partial(
    pl.kernel,
    out_type=jax.ShapeDtypeStruct((_B, _N_EMBD), jnp.float32),
    mesh=plsc.VectorSubcoreMesh(core_axis_name="c", subcore_axis_name="s"),
    compiler_params=pltpu.CompilerParams(use_tc_tiling_on_sc=False),
    scratch_types=[
        pltpu.VMEM((_CHUNK,), jnp.int32),
        pltpu.VMEM((_CHUNK, _N_EMBD), jnp.float32),
        pltpu.SemaphoreType.DMA,
    ],
)
def _embed(idx_hbm, table_hbm, out_hbm, idx_v, rows_v, sem):
    wid = lax.axis_index("s") * _NC + lax.axis_index("c")
    base = wid * _PER_W

    def body(i, carry):
        off = base + i * _CHUNK
        pltpu.sync_copy(idx_hbm.at[pl.ds(off, _CHUNK)], idx_v)
        pltpu.async_copy(table_hbm.at[idx_v], rows_v, sem).wait()
        pltpu.sync_copy(rows_v, out_hbm.at[pl.ds(off, _CHUNK)])
        return carry

    lax.fori_loop(0, _N_CHUNKS, body, 0)


def kernel(tokens, table):
    idx = tokens.reshape(-1).astype(jnp.int32)
    out = _embed(idx, table)
    return out.reshape(_BATCH, _SEQ, _N_EMBD)


# speedup vs baseline: 1.0179x; 1.0179x over previous
import functools

import jax
import jax.numpy as jnp
from jax import lax
from jax.experimental import pallas as pl
from jax.experimental.pallas import tpu as pltpu
from jax.experimental.pallas import tpu_sc as plsc

_VOCAB = 1_000_000
_N_EMBD = 64
_BATCH = 4096
_SEQ = 200
_B = _BATCH * _SEQ
_NC, _NS = 2, 16
_NW = _NC * _NS
_PER_W = _B // _NW
_CHUNK = 512
_N_CHUNKS = _PER_W // _CHUNK
_NBUF = 3


@functools.partial(
    pl.kernel,
    out_type=jax.ShapeDtypeStruct((_B, _N_EMBD), jnp.float32),
    mesh=plsc.VectorSubcoreMesh(core_axis_name="c", subcore_axis_name="s"),
    compiler_params=pltpu.CompilerParams(use_tc_tiling_on_sc=False),
    scratch_types=[
        pltpu.VMEM((_PER_W,), jnp.int32),
        pltpu.VMEM((_NBUF, _CHUNK, _N_EMBD), jnp.float32),
        pltpu.SemaphoreType.DMA((_NBUF,)),
        pltpu.SemaphoreType.DMA((_NBUF,)),
    ],
)
def _embed(idx_hbm, table_hbm, out_hbm, idx_v, rows_v, gsem, ssem):
    wid = lax.axis_index("s") * _NC + lax.axis_index("c")
    base = wid * _PER_W

    pltpu.sync_copy(idx_hbm.at[pl.ds(base, _PER_W)], idx_v)

    def start_gather(j, b):
        pltpu.async_copy(
            table_hbm.at[idx_v.at[pl.ds(j * _CHUNK, _CHUNK)]],
            rows_v.at[b], gsem.at[b])

    for b in range(_NBUF - 1):
        start_gather(b, b)

    def step(i, carry):
        b = lax.rem(i, _NBUF)
        j = i + (_NBUF - 1)
        b2 = lax.rem(j, _NBUF)

        @pl.when(j < _N_CHUNKS)
        def _():
            @pl.when(i >= 1)
            def _():
                pltpu.make_async_copy(
                    rows_v.at[b2],
                    out_hbm.at[pl.ds(base, _CHUNK)],
                    ssem.at[b2]).wait()
            start_gather(j, b2)

        pltpu.make_async_copy(
            table_hbm.at[idx_v.at[pl.ds(0, _CHUNK)]],
            rows_v.at[b], gsem.at[b]).wait()
        pltpu.async_copy(
            rows_v.at[b],
            out_hbm.at[pl.ds(base + i * _CHUNK, _CHUNK)],
            ssem.at[b])
        return carry

    lax.fori_loop(0, _N_CHUNKS, step, 0)

    for b in range(_NBUF):
        pltpu.make_async_copy(
            rows_v.at[b], out_hbm.at[pl.ds(base, _CHUNK)], ssem.at[b]).wait()


def kernel(tokens, table):
    idx = tokens.reshape(-1).astype(jnp.int32)
    out = _embed(idx, table)
    return out.reshape(_BATCH, _SEQ, _N_EMBD)
